# baseline (speedup 1.0000x reference)
import functools

import jax
import jax.numpy as jnp
from jax.experimental import pallas as pl
from jax.experimental.pallas import tpu as pltpu
from jax.experimental.pallas import tpu_sc as plsc

NUM_SPARSE = 26
VOCAB = 100000
EMB = 32
DENSE = 13

GATHER_WINDOW = 128
BLOCK_B = 512
SUB = 4
LANES = 128


def _sc_gather(flat_tables, flat_idx):
    n_idx = flat_idx.shape[0]
    mesh = plsc.VectorSubcoreMesh(core_axis_name="core", subcore_axis_name="subcore")
    idx2d = flat_idx.reshape(1, n_idx)

    @functools.partial(
        pl.kernel,
        out_type=jax.ShapeDtypeStruct((n_idx, EMB), jnp.float32),
        mesh=mesh,
        compiler_params=pltpu.CompilerParams(use_tc_tiling_on_sc=False),
    )
    def gather_kernel(tab_hbm, idx_hbm, out_hbm):
        def body(i_vmem, o_vmem):
            pltpu.sync_copy(tab_hbm.at[i_vmem.at[0]], o_vmem)

        pltpu.emit_pipeline(
            body,
            grid=(n_idx // GATHER_WINDOW,),
            in_specs=[pl.BlockSpec((1, GATHER_WINDOW), lambda i: (0, i))],
            out_specs=[pl.BlockSpec((GATHER_WINDOW, EMB), lambda i: (i, 0))],
            core_axis_name=("core", "subcore"),
            dimension_semantics=(pltpu.PARALLEL,),
        )(idx_hbm, out_hbm)

    return gather_kernel(flat_tables, idx2d)


def _dense_body(cfT_ref, g_ref, bw0T_ref, bb0_ref, bw1T_ref, bb1_ref,
                bw2T_ref, bb2_ref, tw0T_ref, tb0_ref, tw1T_ref, tb1_ref,
                tw2T_ref, tb2_ref, out_ref):
    f32 = jnp.float32

    def mm(wT_ref, x):
        return jnp.dot(wT_ref[...], x, preferred_element_type=f32,
                       precision=jax.lax.Precision.HIGHEST)

    gT = [jnp.transpose(g_ref[t]) for t in range(NUM_SPARSE)]

    for k in range(SUB):
        cf_k = cfT_ref[:, k * LANES:(k + 1) * LANES]
        h = jnp.maximum(mm(bw0T_ref, cf_k) + bb0_ref[...], 0.0)
        h = jnp.maximum(mm(bw1T_ref, h) + bb1_ref[...], 0.0)
        dT = jnp.maximum(mm(bw2T_ref, h) + bb2_ref[...], 0.0)

        S = jnp.stack(
            [dT] + [gT[t][k * EMB:(k + 1) * EMB, :] for t in range(NUM_SPARSE)],
            axis=0)

        cross = []
        for i in range(NUM_SPARSE):
            cross.append(jnp.sum(S[i][None, :, :] * S[i + 1:], axis=1))

        xT = jnp.concatenate([dT] + cross, axis=0)

        h = jnp.maximum(mm(tw0T_ref, xT) + tb0_ref[...], 0.0)
        h = jnp.maximum(mm(tw1T_ref, h) + tb1_ref[...], 0.0)
        out_ref[k:k + 1, :] = mm(tw2T_ref, h) + tb2_ref[...]


def _dense_forward(cfT, gpacked, wts, batch, interpret=False):
    (bw0T, bb0, bw1T, bb1, bw2T, bb2, tw0T, tb0, tw1T, tb1, tw2T, tb2) = wts
    grid = batch // BLOCK_B
    qblk = BLOCK_B // SUB

    def full(a):
        return pl.BlockSpec(a.shape, lambda i: (0,) * a.ndim)

    return pl.pallas_call(
        _dense_body,
        grid=(grid,),
        in_specs=[
            pl.BlockSpec((DENSE, BLOCK_B), lambda i: (0, i)),
            pl.BlockSpec((NUM_SPARSE, qblk, SUB * EMB), lambda i: (0, i, 0)),
            full(bw0T), full(bb0), full(bw1T), full(bb1),
            full(bw2T), full(bb2), full(tw0T), full(tb0),
            full(tw1T), full(tb1), full(tw2T), full(tb2),
        ],
        out_specs=pl.BlockSpec((SUB, qblk), lambda i: (0, i)),
        out_shape=jax.ShapeDtypeStruct((SUB, batch // SUB), jnp.float32),
        compiler_params=pltpu.CompilerParams(
            dimension_semantics=("arbitrary",)),
        interpret=interpret,
    )(cfT, gpacked, bw0T, bb0, bw1T, bb1, bw2T, bb2,
      tw0T, tb0, tw1T, tb1, tw2T, tb2)


def kernel(count_features, category_features, tables, bw0, bb0, bw1, bb1,
           bw2, bb2, tw0, tb0, tw1, tb1, tw2, tb2):
    batch = count_features.shape[0]
    nblk = batch // BLOCK_B

    flat_tables = tables.reshape(NUM_SPARSE * VOCAB, EMB)
    offs = (jnp.arange(NUM_SPARSE, dtype=jnp.int32) * VOCAB)[:, None]
    flat_idx = (category_features.T.astype(jnp.int32) + offs).reshape(-1)
    gathered = _sc_gather(flat_tables, flat_idx)
    gpacked = gathered.reshape(NUM_SPARSE, batch // SUB, SUB * EMB)

    cfT = count_features.T.reshape(DENSE, nblk, BLOCK_B // SUB, SUB)
    cfT = cfT.transpose(0, 1, 3, 2).reshape(DENSE, batch)

    wts = (bw0.T, bb0[:, None], bw1.T, bb1[:, None], bw2.T, bb2[:, None],
           tw0.T, tb0[:, None], tw1.T, tb1[:, None], tw2.T, tb2[:, None])
    out = _dense_forward(cfT, gpacked, wts, batch)
    return out.T.reshape(batch, 1)


# speedup vs baseline: 1.1954x; 1.1954x over previous
import functools

import jax
import jax.numpy as jnp
from jax.experimental import pallas as pl
from jax.experimental.pallas import tpu as pltpu
from jax.experimental.pallas import tpu_sc as plsc

NUM_SPARSE = 26
VOCAB = 100000
EMB = 32
DENSE = 13

GATHER_WINDOW = 512
BLOCK_B = 512
SUB = 4
LANES = 128


def _sc_gather(flat_tables, flat_idx):
    n_idx = flat_idx.shape[0]
    mesh = plsc.VectorSubcoreMesh(core_axis_name="core", subcore_axis_name="subcore")
    idx2d = flat_idx.reshape(1, n_idx)

    @functools.partial(
        pl.kernel,
        out_type=jax.ShapeDtypeStruct((n_idx, EMB), jnp.float32),
        mesh=mesh,
        compiler_params=pltpu.CompilerParams(use_tc_tiling_on_sc=False),
    )
    def gather_kernel(tab_hbm, idx_hbm, out_hbm):
        def body(i_vmem, o_vmem):
            pltpu.sync_copy(tab_hbm.at[i_vmem.at[0]], o_vmem)

        pltpu.emit_pipeline(
            body,
            grid=(n_idx // GATHER_WINDOW,),
            in_specs=[pl.BlockSpec((1, GATHER_WINDOW), lambda i: (0, i))],
            out_specs=[pl.BlockSpec((GATHER_WINDOW, EMB), lambda i: (i, 0))],
            core_axis_name=("core", "subcore"),
            dimension_semantics=(pltpu.PARALLEL,),
        )(idx_hbm, out_hbm)

    return gather_kernel(flat_tables, idx2d)


def _dense_body(cfT_ref, g_ref, bw0T_ref, bb0_ref, bw1T_ref, bb1_ref,
                bw2T_ref, bb2_ref, tw0T_ref, tb0_ref, tw1T_ref, tb1_ref,
                tw2T_ref, tb2_ref, out_ref):
    f32 = jnp.float32

    def mm(wT_ref, x):
        return jnp.dot(wT_ref[...], x, preferred_element_type=f32,
                       precision=jax.lax.Precision.HIGHEST)

    gT = []
    for t in range(NUM_SPARSE):
        gt = jnp.transpose(g_ref[t])
        gT.append(jnp.concatenate(
            [gt[k * EMB:(k + 1) * EMB, :] for k in range(SUB)], axis=1))

    h = jnp.maximum(mm(bw0T_ref, cfT_ref[...]) + bb0_ref[...], 0.0)
    h = jnp.maximum(mm(bw1T_ref, h) + bb1_ref[...], 0.0)
    dT = jnp.maximum(mm(bw2T_ref, h) + bb2_ref[...], 0.0)

    S = jnp.stack([dT] + gT, axis=0)

    cross = []
    for i in range(NUM_SPARSE):
        cross.append(jnp.sum(S[i][None, :, :] * S[i + 1:], axis=1))

    xT = jnp.concatenate([dT] + cross, axis=0)

    h = jnp.maximum(mm(tw0T_ref, xT) + tb0_ref[...], 0.0)
    h = jnp.maximum(mm(tw1T_ref, h) + tb1_ref[...], 0.0)
    y = mm(tw2T_ref, h) + tb2_ref[...]
    out_ref[...] = y.reshape(SUB, BLOCK_B // SUB)


def _dense_forward(cfT, gpacked, wts, batch, interpret=False):
    (bw0T, bb0, bw1T, bb1, bw2T, bb2, tw0T, tb0, tw1T, tb1, tw2T, tb2) = wts
    grid = batch // BLOCK_B
    qblk = BLOCK_B // SUB

    def full(a):
        return pl.BlockSpec(a.shape, lambda i: (0,) * a.ndim)

    return pl.pallas_call(
        _dense_body,
        grid=(grid,),
        in_specs=[
            pl.BlockSpec((DENSE, BLOCK_B), lambda i: (0, i)),
            pl.BlockSpec((NUM_SPARSE, qblk, SUB * EMB), lambda i: (0, i, 0)),
            full(bw0T), full(bb0), full(bw1T), full(bb1),
            full(bw2T), full(bb2), full(tw0T), full(tb0),
            full(tw1T), full(tb1), full(tw2T), full(tb2),
        ],
        out_specs=pl.BlockSpec((SUB, qblk), lambda i: (0, i)),
        out_shape=jax.ShapeDtypeStruct((SUB, batch // SUB), jnp.float32),
        compiler_params=pltpu.CompilerParams(
            dimension_semantics=("parallel",)),
        interpret=interpret,
    )(cfT, gpacked, bw0T, bb0, bw1T, bb1, bw2T, bb2,
      tw0T, tb0, tw1T, tb1, tw2T, tb2)


def kernel(count_features, category_features, tables, bw0, bb0, bw1, bb1,
           bw2, bb2, tw0, tb0, tw1, tb1, tw2, tb2):
    batch = count_features.shape[0]
    nblk = batch // BLOCK_B

    flat_tables = tables.reshape(NUM_SPARSE * VOCAB, EMB)
    offs = (jnp.arange(NUM_SPARSE, dtype=jnp.int32) * VOCAB)[:, None]
    flat_idx = (category_features.T.astype(jnp.int32) + offs).reshape(-1)
    gathered = _sc_gather(flat_tables, flat_idx)
    gpacked = gathered.reshape(NUM_SPARSE, batch // SUB, SUB * EMB)

    cfT = count_features.T.reshape(DENSE, nblk, BLOCK_B // SUB, SUB)
    cfT = cfT.transpose(0, 1, 3, 2).reshape(DENSE, batch)

    wts = (bw0.T, bb0[:, None], bw1.T, bb1[:, None], bw2.T, bb2[:, None],
           tw0.T, tb0[:, None], tw1.T, tb1[:, None], tw2.T, tb2[:, None])
    out = _dense_forward(cfT, gpacked, wts, batch)
    return out.T.reshape(batch, 1)
